# Initial kernel scaffold; baseline (speedup 1.0000x reference)
import jax
import jax.numpy as jnp
from jax.experimental import pallas as pl


def kernel(query, value, spatial_shapes, reference_points, query_pos, level_start_index, Wv, bv, Wo, bo, Wa, ba):
    raise NotImplementedError("write your pallas kernel here")


# speedup vs baseline: 2496.1164x; 2496.1164x over previous
import functools

import jax
import jax.numpy as jnp
import numpy as np
from jax import lax
from jax.experimental import pallas as pl
from jax.experimental.pallas import tpu as pltpu
from jax.experimental.pallas import tpu_sc as plsc

_EMBED = 256
_HEADS = 8
_LEVELS = 4
_POINTS = 8
_ZA = 4
_HD = _EMBED // _HEADS
_SHAPES = ((116, 200), (58, 100), (29, 50), (15, 25))
_NV = sum(h * w for h, w in _SHAPES)
_K = _LEVELS * _POINTS
_KC = 4 * _K

_VBLK = 1024
_QBLK = 1000


def _col_consts():
    k = np.arange(_HEADS * _K)
    h = k // _K
    l = (k % _K) // _POINTS
    p = k % _POINTS
    z = p % _ZA
    wf = np.array([s[1] for s in _SHAPES], np.float32)[l]
    hf = np.array([s[0] for s in _SHAPES], np.float32)[l]
    sizes = [s[0] * s[1] for s in _SHAPES]
    offs = np.concatenate([[0], np.cumsum(sizes)[:-1]]).astype(np.int32)
    ls = offs[l]
    sx = np.zeros((2 * _ZA, _HEADS * _K), np.float32)
    sy = np.zeros((2 * _ZA, _HEADS * _K), np.float32)
    sx[2 * z, k] = 1.0
    sy[2 * z + 1, k] = 1.0
    ones = (k[:, None] // _K == k[None, :] // _K).astype(np.float32)
    return (wf.astype(np.float32), hf.astype(np.float32), ls, h.astype(np.int32),
            wf.astype(np.int32), hf.astype(np.int32), sx, sy, ones)


def _vproj_body(val_ref, wv_ref, bv_ref, out_ref):
    out_ref[...] = (
        jnp.dot(val_ref[...], wv_ref[...], preferred_element_type=jnp.float32, precision=jax.lax.Precision.HIGHEST)
        + bv_ref[...]
    )


def _sample_body(q_ref, qp_ref, rp_ref, wox_ref, box_ref, woy_ref, boy_ref,
                 wa_ref, ba_ref, ones_ref, sx_ref, sy_ref, wcol_ref, hcol_ref,
                 ls_ref, hid_ref, wi_ref, hi_ref, idx_ref, w_ref):
    q = q_ref[...] + qp_ref[...]
    sox = jnp.dot(q, wox_ref[...], preferred_element_type=jnp.float32, precision=jax.lax.Precision.HIGHEST) + box_ref[...]
    soy = jnp.dot(q, woy_ref[...], preferred_element_type=jnp.float32, precision=jax.lax.Precision.HIGHEST) + boy_ref[...]
    al = jnp.dot(q, wa_ref[...], preferred_element_type=jnp.float32, precision=jax.lax.Precision.HIGHEST) + ba_ref[...]
    m = jnp.max(al, axis=-1, keepdims=True)
    e = jnp.exp(al - m)
    denom = jnp.dot(e, ones_ref[...], preferred_element_type=jnp.float32, precision=jax.lax.Precision.HIGHEST)
    aw = e / denom
    rpx = jnp.dot(rp_ref[...], sx_ref[...], preferred_element_type=jnp.float32, precision=jax.lax.Precision.HIGHEST)
    rpy = jnp.dot(rp_ref[...], sy_ref[...], preferred_element_type=jnp.float32, precision=jax.lax.Precision.HIGHEST)
    px = rpx * wcol_ref[...] + sox - 0.5
    py = rpy * hcol_ref[...] + soy - 0.5
    x0 = jnp.floor(px)
    y0 = jnp.floor(py)
    fx = px - x0
    fy = py - y0
    x0i = x0.astype(jnp.int32)
    y0i = y0.astype(jnp.int32)
    wi = wi_ref[...]
    hi = hi_ref[...]
    ls = ls_ref[...]
    hid = hid_ref[...]
    corners = (
        (0, 0, (1.0 - fx) * (1.0 - fy)),
        (1, 0, fx * (1.0 - fy)),
        (0, 1, (1.0 - fx) * fy),
        (1, 1, fx * fy),
    )
    for c, (dx, dy, wbil) in enumerate(corners):
        xi = x0i + dx
        yi = y0i + dy
        valid = (xi >= 0) & (xi < wi) & (yi >= 0) & (yi < hi)
        xc = jnp.clip(xi, 0, wi - 1)
        yc = jnp.clip(yi, 0, hi - 1)
        gidx = (ls + yc * wi + xc) * _HEADS + hid
        wgt = aw * wbil * valid.astype(jnp.float32)
        for h in range(_HEADS):
            idx_ref[:, h, pl.ds(c * _K, _K)] = gidx[:, h * _K:(h + 1) * _K]
            w_ref[:, h, pl.ds(c * _K, _K)] = wgt[:, h * _K:(h + 1) * _K]


def _sampling_params(query, query_pos, reference_points, Wo, bo, Wa, ba, nq):
    wf, hf, ls, hid, wi, hi, sx, sy, ones = _col_consts()
    wox = Wo[:, 0::2]
    woy = Wo[:, 1::2]
    box = bo[0::2].reshape(1, -1)
    boy = bo[1::2].reshape(1, -1)
    rp_flat = reference_points.reshape(nq, 2 * _ZA)
    n256 = _HEADS * _K
    row = lambda a: jnp.asarray(a).reshape(1, n256)
    grid = pl.cdiv(nq, _QBLK)
    bcast = lambda shape: pl.BlockSpec(shape, lambda i: (0,) * len(shape))
    return pl.pallas_call(
        _sample_body,
        grid=(grid,),
        in_specs=[
            pl.BlockSpec((_QBLK, _EMBED), lambda i: (i, 0)),
            pl.BlockSpec((_QBLK, _EMBED), lambda i: (i, 0)),
            pl.BlockSpec((_QBLK, 2 * _ZA), lambda i: (i, 0)),
            bcast((_EMBED, n256)), bcast((1, n256)),
            bcast((_EMBED, n256)), bcast((1, n256)),
            bcast((_EMBED, n256)), bcast((1, n256)),
            bcast((n256, n256)),
            bcast((2 * _ZA, n256)), bcast((2 * _ZA, n256)),
            bcast((1, n256)), bcast((1, n256)),
            bcast((1, n256)), bcast((1, n256)),
            bcast((1, n256)), bcast((1, n256)),
        ],
        out_specs=[
            pl.BlockSpec((_QBLK, _HEADS, _KC), lambda i: (i, 0, 0)),
            pl.BlockSpec((_QBLK, _HEADS, _KC), lambda i: (i, 0, 0)),
        ],
        out_shape=[
            jax.ShapeDtypeStruct((nq, _HEADS, _KC), jnp.int32),
            jax.ShapeDtypeStruct((nq, _HEADS, _KC), jnp.float32),
        ],
    )(query, query_pos, rp_flat, wox, box, woy, boy,
      jnp.asarray(Wa), jnp.asarray(ba).reshape(1, n256), jnp.asarray(ones),
      jnp.asarray(sx), jnp.asarray(sy), row(wf), row(hf),
      row(ls), row(hid), row(wi), row(hi))


_GB = 10


def _gather_reduce(idx, w, table, npair):
    info = plsc.get_sparse_core_info()
    nw = info.num_cores * info.num_subcores
    per_w = npair // nw
    nb = per_w // _GB
    mesh = plsc.VectorSubcoreMesh(
        core_axis_name="c", subcore_axis_name="s",
        num_cores=info.num_cores, num_subcores=info.num_subcores)

    @functools.partial(
        pl.kernel,
        out_type=jax.ShapeDtypeStruct((npair, _HD), jnp.float32),
        mesh=mesh,
        scratch_types=[
            pltpu.VMEM((_GB, _KC), jnp.int32),
            pltpu.VMEM((_GB, _KC), jnp.float32),
            pltpu.VMEM((_GB, _KC, _HD), jnp.float32),
            pltpu.VMEM((_GB, _HD), jnp.float32),
            pltpu.SemaphoreType.DMA,
        ],
        compiler_params=pltpu.CompilerParams(use_tc_tiling_on_sc=False),
    )
    def run(idx_hbm, w_hbm, table_hbm, out_hbm, idx_v, w_v, rows_v, out_v, sem):
        wid = lax.axis_index("s") * info.num_cores + lax.axis_index("c")
        base = wid * per_w

        def batch(b, _):
            r0 = base + b * _GB
            pltpu.sync_copy(idx_hbm.at[pl.ds(r0, _GB)], idx_v)
            pltpu.sync_copy(w_hbm.at[pl.ds(r0, _GB)], w_v)
            copies = [
                pltpu.async_copy(
                    table_hbm.at[idx_v.at[g]], rows_v.at[g], sem)
                for g in range(_GB)
            ]
            for c in copies:
                c.wait()
            for g in range(_GB):
                rows = rows_v.at[g]

                def cstep(c, acc):
                    wvec = w_v[g, pl.ds(c * 16, 16)]
                    a0, a1 = acc
                    for j in range(16):
                        s = wvec[j]
                        k = c * 16 + j
                        a0 = a0 + rows[k, pl.ds(0, 16)] * s
                        a1 = a1 + rows[k, pl.ds(16, 16)] * s
                    return (a0, a1)

                a0, a1 = lax.fori_loop(
                    0, _KC // 16, cstep,
                    (jnp.zeros((16,), jnp.float32), jnp.zeros((16,), jnp.float32)))
                out_v[g, pl.ds(0, 16)] = a0
                out_v[g, pl.ds(16, 16)] = a1
            pltpu.sync_copy(out_v, out_hbm.at[pl.ds(r0, _GB)])
            return 0

        lax.fori_loop(0, nb, batch, 0)

    return run(idx, w, table)


def kernel(query, value, spatial_shapes, reference_points, query_pos,
           level_start_index, Wv, bv, Wo, bo, Wa, ba):
    del spatial_shapes, level_start_index
    bs, nq, _ = query.shape
    nv = value.shape[1]
    q2 = query.reshape(nq, _EMBED)
    qp2 = query_pos.reshape(nq, _EMBED)

    table = pl.pallas_call(
        _vproj_body,
        grid=(pl.cdiv(nv, _VBLK),),
        in_specs=[
            pl.BlockSpec((_VBLK, _EMBED), lambda i: (i, 0)),
            pl.BlockSpec((_EMBED, _EMBED), lambda i: (0, 0)),
            pl.BlockSpec((1, _EMBED), lambda i: (0, 0)),
        ],
        out_specs=pl.BlockSpec((_VBLK, _EMBED), lambda i: (i, 0)),
        out_shape=jax.ShapeDtypeStruct((nv, _EMBED), jnp.float32),
    )(value.reshape(nv, _EMBED), Wv, bv.reshape(1, _EMBED))
    table = table.reshape(nv * _HEADS, _HD)

    idx, w = _sampling_params(q2, qp2, reference_points, Wo, bo, Wa, ba, nq)
    npair = nq * _HEADS
    idx = idx.reshape(npair, _KC)
    w = w.reshape(npair, _KC)

    out = _gather_reduce(idx, w, table, npair)
    return out.reshape(bs, nq, _EMBED)


# speedup vs baseline: 2824.3025x; 1.1315x over previous
import functools

import jax
import jax.numpy as jnp
import numpy as np
from jax import lax
from jax.experimental import pallas as pl
from jax.experimental.pallas import tpu as pltpu
from jax.experimental.pallas import tpu_sc as plsc

_EMBED = 256
_HEADS = 8
_LEVELS = 4
_POINTS = 8
_ZA = 4
_HD = _EMBED // _HEADS
_SHAPES = ((116, 200), (58, 100), (29, 50), (15, 25))
_NV = sum(h * w for h, w in _SHAPES)
_K = _LEVELS * _POINTS
_KC = 4 * _K

_VBLK = 1024
_QBLK = 1000


def _col_consts():
    k = np.arange(_HEADS * _K)
    h = k // _K
    l = (k % _K) // _POINTS
    p = k % _POINTS
    z = p % _ZA
    wf = np.array([s[1] for s in _SHAPES], np.float32)[l]
    hf = np.array([s[0] for s in _SHAPES], np.float32)[l]
    sizes = [s[0] * s[1] for s in _SHAPES]
    offs = np.concatenate([[0], np.cumsum(sizes)[:-1]]).astype(np.int32)
    ls = offs[l]
    sx = np.zeros((2 * _ZA, _HEADS * _K), np.float32)
    sy = np.zeros((2 * _ZA, _HEADS * _K), np.float32)
    sx[2 * z, k] = 1.0
    sy[2 * z + 1, k] = 1.0
    ones = (k[:, None] // _K == k[None, :] // _K).astype(np.float32)
    return (wf.astype(np.float32), hf.astype(np.float32), ls, h.astype(np.int32),
            wf.astype(np.int32), hf.astype(np.int32), sx, sy, ones)


def _vproj_body(val_ref, wv_ref, bv_ref, out_ref):
    out_ref[...] = (
        jnp.dot(val_ref[...], wv_ref[...], preferred_element_type=jnp.float32, precision=jax.lax.Precision.HIGHEST)
        + bv_ref[...]
    )


def _sample_body(q_ref, qp_ref, rp_ref, wox_ref, box_ref, woy_ref, boy_ref,
                 wa_ref, ba_ref, ones_ref, sx_ref, sy_ref, wcol_ref, hcol_ref,
                 ls_ref, hid_ref, wi_ref, hi_ref, idx_ref, w_ref):
    q = q_ref[...] + qp_ref[...]
    sox = jnp.dot(q, wox_ref[...], preferred_element_type=jnp.float32, precision=jax.lax.Precision.HIGHEST) + box_ref[...]
    soy = jnp.dot(q, woy_ref[...], preferred_element_type=jnp.float32, precision=jax.lax.Precision.HIGHEST) + boy_ref[...]
    al = jnp.dot(q, wa_ref[...], preferred_element_type=jnp.float32, precision=jax.lax.Precision.HIGHEST) + ba_ref[...]
    m = jnp.max(al, axis=-1, keepdims=True)
    e = jnp.exp(al - m)
    denom = jnp.dot(e, ones_ref[...], preferred_element_type=jnp.float32, precision=jax.lax.Precision.HIGHEST)
    aw = e / denom
    rpx = jnp.dot(rp_ref[...], sx_ref[...], preferred_element_type=jnp.float32, precision=jax.lax.Precision.HIGHEST)
    rpy = jnp.dot(rp_ref[...], sy_ref[...], preferred_element_type=jnp.float32, precision=jax.lax.Precision.HIGHEST)
    px = rpx * wcol_ref[...] + sox - 0.5
    py = rpy * hcol_ref[...] + soy - 0.5
    x0 = jnp.floor(px)
    y0 = jnp.floor(py)
    fx = px - x0
    fy = py - y0
    x0i = x0.astype(jnp.int32)
    y0i = y0.astype(jnp.int32)
    wi = wi_ref[...]
    hi = hi_ref[...]
    ls = ls_ref[...]
    hid = hid_ref[...]
    corners = (
        (0, 0, (1.0 - fx) * (1.0 - fy)),
        (1, 0, fx * (1.0 - fy)),
        (0, 1, (1.0 - fx) * fy),
        (1, 1, fx * fy),
    )
    for c, (dx, dy, wbil) in enumerate(corners):
        xi = x0i + dx
        yi = y0i + dy
        valid = (xi >= 0) & (xi < wi) & (yi >= 0) & (yi < hi)
        xc = jnp.clip(xi, 0, wi - 1)
        yc = jnp.clip(yi, 0, hi - 1)
        gidx = (ls + yc * wi + xc) * _HEADS + hid
        wgt = aw * wbil * valid.astype(jnp.float32)
        for h in range(_HEADS):
            idx_ref[:, h, pl.ds(c * _K, _K)] = gidx[:, h * _K:(h + 1) * _K]
            w_ref[:, h, pl.ds(c * _K, _K)] = wgt[:, h * _K:(h + 1) * _K]


def _sampling_params(query, query_pos, reference_points, Wo, bo, Wa, ba, nq):
    wf, hf, ls, hid, wi, hi, sx, sy, ones = _col_consts()
    wox = Wo[:, 0::2]
    woy = Wo[:, 1::2]
    box = bo[0::2].reshape(1, -1)
    boy = bo[1::2].reshape(1, -1)
    rp_flat = reference_points.reshape(nq, 2 * _ZA)
    n256 = _HEADS * _K
    row = lambda a: jnp.asarray(a).reshape(1, n256)
    grid = pl.cdiv(nq, _QBLK)
    bcast = lambda shape: pl.BlockSpec(shape, lambda i: (0,) * len(shape))
    return pl.pallas_call(
        _sample_body,
        grid=(grid,),
        in_specs=[
            pl.BlockSpec((_QBLK, _EMBED), lambda i: (i, 0)),
            pl.BlockSpec((_QBLK, _EMBED), lambda i: (i, 0)),
            pl.BlockSpec((_QBLK, 2 * _ZA), lambda i: (i, 0)),
            bcast((_EMBED, n256)), bcast((1, n256)),
            bcast((_EMBED, n256)), bcast((1, n256)),
            bcast((_EMBED, n256)), bcast((1, n256)),
            bcast((n256, n256)),
            bcast((2 * _ZA, n256)), bcast((2 * _ZA, n256)),
            bcast((1, n256)), bcast((1, n256)),
            bcast((1, n256)), bcast((1, n256)),
            bcast((1, n256)), bcast((1, n256)),
        ],
        out_specs=[
            pl.BlockSpec((_QBLK, _HEADS, _KC), lambda i: (i, 0, 0)),
            pl.BlockSpec((_QBLK, _HEADS, _KC), lambda i: (i, 0, 0)),
        ],
        out_shape=[
            jax.ShapeDtypeStruct((nq, _HEADS, _KC), jnp.int32),
            jax.ShapeDtypeStruct((nq, _HEADS, _KC), jnp.float32),
        ],
    )(query, query_pos, rp_flat, wox, box, woy, boy,
      jnp.asarray(Wa), jnp.asarray(ba).reshape(1, n256), jnp.asarray(ones),
      jnp.asarray(sx), jnp.asarray(sy), row(wf), row(hf),
      row(ls), row(hid), row(wi), row(hi))


_GB = 10


def _gather_reduce(idx, w, table, npair):
    info = plsc.get_sparse_core_info()
    nw = info.num_cores * info.num_subcores
    per_w = npair // nw
    nb = per_w // _GB
    mesh = plsc.VectorSubcoreMesh(
        core_axis_name="c", subcore_axis_name="s",
        num_cores=info.num_cores, num_subcores=info.num_subcores)

    @functools.partial(
        pl.kernel,
        out_type=jax.ShapeDtypeStruct((npair, _HD), jnp.float32),
        mesh=mesh,
        scratch_types=[
            pltpu.VMEM((2, _GB, _KC), jnp.int32),
            pltpu.VMEM((2, _GB, _KC), jnp.float32),
            pltpu.VMEM((2, _GB, _KC, _HD), jnp.float32),
            pltpu.VMEM((2, _GB, _HD), jnp.float32),
            pltpu.SemaphoreType.DMA,
            pltpu.SemaphoreType.DMA,
        ],
        compiler_params=pltpu.CompilerParams(use_tc_tiling_on_sc=False),
    )
    def run(idx_hbm, w_hbm, table_hbm, out_hbm, idx_v, w_v, rows_v, out_v,
            sem0, sem1):
        wid = lax.axis_index("s") * info.num_cores + lax.axis_index("c")
        base = wid * per_w
        sems = (sem0, sem1)

        def fire(b, par):
            r0 = base + b * _GB
            pltpu.sync_copy(idx_hbm.at[pl.ds(r0, _GB)], idx_v.at[par])
            pltpu.sync_copy(w_hbm.at[pl.ds(r0, _GB)], w_v.at[par])
            for g in range(_GB):
                pltpu.async_copy(
                    table_hbm.at[idx_v.at[par, g]], rows_v.at[par, g], sems[par])

        def drain(par):
            for g in range(_GB):
                pltpu.make_async_copy(
                    table_hbm.at[pl.ds(0, _KC)], rows_v.at[par, g],
                    sems[par]).wait()

        def compute_store(b, par):
            r0 = base + b * _GB
            for g in range(_GB):
                rows = rows_v.at[par, g]

                def cstep(c, acc):
                    wvec = w_v[par, g, pl.ds(c * 16, 16)]
                    a0, a1 = acc
                    for j in range(16):
                        s = wvec[j]
                        k = c * 16 + j
                        a0 = a0 + rows[k, pl.ds(0, 16)] * s
                        a1 = a1 + rows[k, pl.ds(16, 16)] * s
                    return (a0, a1)

                a0, a1 = lax.fori_loop(
                    0, _KC // 16, cstep,
                    (jnp.zeros((16,), jnp.float32), jnp.zeros((16,), jnp.float32)))
                out_v[par, g, pl.ds(0, 16)] = a0
                out_v[par, g, pl.ds(16, 16)] = a1
            pltpu.sync_copy(out_v.at[par], out_hbm.at[pl.ds(r0, _GB)])

        fire(0, 0)
        nouter = nb // 2

        def outer(i, _):
            b0 = 2 * i
            fire(b0 + 1, 1)
            drain(0)
            compute_store(b0, 0)

            @pl.when(i + 1 < nouter)
            def _():
                fire(b0 + 2, 0)

            drain(1)
            compute_store(b0 + 1, 1)
            return 0

        lax.fori_loop(0, nouter, outer, 0)

    return run(idx, w, table)


def kernel(query, value, spatial_shapes, reference_points, query_pos,
           level_start_index, Wv, bv, Wo, bo, Wa, ba):
    del spatial_shapes, level_start_index
    bs, nq, _ = query.shape
    nv = value.shape[1]
    q2 = query.reshape(nq, _EMBED)
    qp2 = query_pos.reshape(nq, _EMBED)

    table = pl.pallas_call(
        _vproj_body,
        grid=(pl.cdiv(nv, _VBLK),),
        in_specs=[
            pl.BlockSpec((_VBLK, _EMBED), lambda i: (i, 0)),
            pl.BlockSpec((_EMBED, _EMBED), lambda i: (0, 0)),
            pl.BlockSpec((1, _EMBED), lambda i: (0, 0)),
        ],
        out_specs=pl.BlockSpec((_VBLK, _EMBED), lambda i: (i, 0)),
        out_shape=jax.ShapeDtypeStruct((nv, _EMBED), jnp.float32),
    )(value.reshape(nv, _EMBED), Wv, bv.reshape(1, _EMBED))
    table = table.reshape(nv * _HEADS, _HD)

    idx, w = _sampling_params(q2, qp2, reference_points, Wo, bo, Wa, ba, nq)
    npair = nq * _HEADS
    idx = idx.reshape(npair, _KC)
    w = w.reshape(npair, _KC)

    out = _gather_reduce(idx, w, table, npair)
    return out.reshape(bs, nq, _EMBED)


# speedup vs baseline: 2832.1091x; 1.0028x over previous
import functools

import jax
import jax.numpy as jnp
import numpy as np
from jax import lax
from jax.experimental import pallas as pl
from jax.experimental.pallas import tpu as pltpu
from jax.experimental.pallas import tpu_sc as plsc

_EMBED = 256
_HEADS = 8
_LEVELS = 4
_POINTS = 8
_ZA = 4
_HD = _EMBED // _HEADS
_SHAPES = ((116, 200), (58, 100), (29, 50), (15, 25))
_NV = sum(h * w for h, w in _SHAPES)
_K = _LEVELS * _POINTS
_KC = 4 * _K

_VBLK = 1024
_QBLK = 1000


def _col_consts():
    k = np.arange(_HEADS * _K)
    h = k // _K
    l = (k % _K) // _POINTS
    p = k % _POINTS
    z = p % _ZA
    wf = np.array([s[1] for s in _SHAPES], np.float32)[l]
    hf = np.array([s[0] for s in _SHAPES], np.float32)[l]
    sizes = [s[0] * s[1] for s in _SHAPES]
    offs = np.concatenate([[0], np.cumsum(sizes)[:-1]]).astype(np.int32)
    ls = offs[l]
    sx = np.zeros((2 * _ZA, _HEADS * _K), np.float32)
    sy = np.zeros((2 * _ZA, _HEADS * _K), np.float32)
    sx[2 * z, k] = 1.0
    sy[2 * z + 1, k] = 1.0
    ones = (k[:, None] // _K == k[None, :] // _K).astype(np.float32)
    return (wf.astype(np.float32), hf.astype(np.float32), ls, h.astype(np.int32),
            wf.astype(np.int32), hf.astype(np.int32), sx, sy, ones)


def _vproj_body(val_ref, wv_ref, bv_ref, out_ref):
    out_ref[...] = (
        jnp.dot(val_ref[...], wv_ref[...], preferred_element_type=jnp.float32, precision=jax.lax.Precision.HIGHEST)
        + bv_ref[...]
    )


def _sample_body(q_ref, qp_ref, rp_ref, wox_ref, box_ref, woy_ref, boy_ref,
                 wa_ref, ba_ref, ones_ref, sx_ref, sy_ref, wcol_ref, hcol_ref,
                 ls_ref, hid_ref, wi_ref, hi_ref, idx_ref, w_ref):
    q = q_ref[...] + qp_ref[...]
    sox = jnp.dot(q, wox_ref[...], preferred_element_type=jnp.float32, precision=jax.lax.Precision.HIGHEST) + box_ref[...]
    soy = jnp.dot(q, woy_ref[...], preferred_element_type=jnp.float32, precision=jax.lax.Precision.HIGHEST) + boy_ref[...]
    al = jnp.dot(q, wa_ref[...], preferred_element_type=jnp.float32, precision=jax.lax.Precision.HIGHEST) + ba_ref[...]
    m = jnp.max(al, axis=-1, keepdims=True)
    e = jnp.exp(al - m)
    denom = jnp.dot(e, ones_ref[...], preferred_element_type=jnp.float32, precision=jax.lax.Precision.HIGHEST)
    aw = e / denom
    rpx = jnp.dot(rp_ref[...], sx_ref[...], preferred_element_type=jnp.float32, precision=jax.lax.Precision.HIGHEST)
    rpy = jnp.dot(rp_ref[...], sy_ref[...], preferred_element_type=jnp.float32, precision=jax.lax.Precision.HIGHEST)
    px = rpx * wcol_ref[...] + sox - 0.5
    py = rpy * hcol_ref[...] + soy - 0.5
    x0 = jnp.floor(px)
    y0 = jnp.floor(py)
    fx = px - x0
    fy = py - y0
    x0i = x0.astype(jnp.int32)
    y0i = y0.astype(jnp.int32)
    wi = wi_ref[...]
    hi = hi_ref[...]
    ls = ls_ref[...]
    hid = hid_ref[...]
    corners = (
        (0, 0, (1.0 - fx) * (1.0 - fy)),
        (1, 0, fx * (1.0 - fy)),
        (0, 1, (1.0 - fx) * fy),
        (1, 1, fx * fy),
    )
    for c, (dx, dy, wbil) in enumerate(corners):
        xi = x0i + dx
        yi = y0i + dy
        valid = (xi >= 0) & (xi < wi) & (yi >= 0) & (yi < hi)
        xc = jnp.clip(xi, 0, wi - 1)
        yc = jnp.clip(yi, 0, hi - 1)
        gidx = (ls + yc * wi + xc) * _HEADS + hid
        wgt = aw * wbil * valid.astype(jnp.float32)
        for h in range(_HEADS):
            idx_ref[:, h, pl.ds(c * _K, _K)] = gidx[:, h * _K:(h + 1) * _K]
            w_ref[:, h, pl.ds(c * _K, _K)] = wgt[:, h * _K:(h + 1) * _K]


def _sampling_params(query, query_pos, reference_points, Wo, bo, Wa, ba, nq):
    wf, hf, ls, hid, wi, hi, sx, sy, ones = _col_consts()
    wox = Wo[:, 0::2]
    woy = Wo[:, 1::2]
    box = bo[0::2].reshape(1, -1)
    boy = bo[1::2].reshape(1, -1)
    rp_flat = reference_points.reshape(nq, 2 * _ZA)
    n256 = _HEADS * _K
    row = lambda a: jnp.asarray(a).reshape(1, n256)
    grid = pl.cdiv(nq, _QBLK)
    bcast = lambda shape: pl.BlockSpec(shape, lambda i: (0,) * len(shape))
    return pl.pallas_call(
        _sample_body,
        grid=(grid,),
        in_specs=[
            pl.BlockSpec((_QBLK, _EMBED), lambda i: (i, 0)),
            pl.BlockSpec((_QBLK, _EMBED), lambda i: (i, 0)),
            pl.BlockSpec((_QBLK, 2 * _ZA), lambda i: (i, 0)),
            bcast((_EMBED, n256)), bcast((1, n256)),
            bcast((_EMBED, n256)), bcast((1, n256)),
            bcast((_EMBED, n256)), bcast((1, n256)),
            bcast((n256, n256)),
            bcast((2 * _ZA, n256)), bcast((2 * _ZA, n256)),
            bcast((1, n256)), bcast((1, n256)),
            bcast((1, n256)), bcast((1, n256)),
            bcast((1, n256)), bcast((1, n256)),
        ],
        out_specs=[
            pl.BlockSpec((_QBLK, _HEADS, _KC), lambda i: (i, 0, 0)),
            pl.BlockSpec((_QBLK, _HEADS, _KC), lambda i: (i, 0, 0)),
        ],
        out_shape=[
            jax.ShapeDtypeStruct((nq, _HEADS, _KC), jnp.int32),
            jax.ShapeDtypeStruct((nq, _HEADS, _KC), jnp.float32),
        ],
    )(query, query_pos, rp_flat, wox, box, woy, boy,
      jnp.asarray(Wa), jnp.asarray(ba).reshape(1, n256), jnp.asarray(ones),
      jnp.asarray(sx), jnp.asarray(sy), row(wf), row(hf),
      row(ls), row(hid), row(wi), row(hi))


_GB = 10


def _gather_reduce(idx, w, table, npair):
    info = plsc.get_sparse_core_info()
    nw = info.num_cores * info.num_subcores
    per_w = npair // nw
    nb = per_w // _GB
    mesh = plsc.VectorSubcoreMesh(
        core_axis_name="c", subcore_axis_name="s",
        num_cores=info.num_cores, num_subcores=info.num_subcores)

    @functools.partial(
        pl.kernel,
        out_type=jax.ShapeDtypeStruct((npair, _HD), jnp.float32),
        mesh=mesh,
        scratch_types=[
            pltpu.VMEM((2, _GB, _KC), jnp.int32),
            pltpu.VMEM((2, _GB, _KC), jnp.float32),
            pltpu.VMEM((2, _GB, _KC, _HD), jnp.float32),
            pltpu.VMEM((2, _GB, _HD), jnp.float32),
            pltpu.SemaphoreType.DMA,
            pltpu.SemaphoreType.DMA,
        ],
        compiler_params=pltpu.CompilerParams(use_tc_tiling_on_sc=False),
    )
    def run(idx_hbm, w_hbm, table_hbm, out_hbm, idx_v, w_v, rows_v, out_v,
            sem0, sem1):
        wid = lax.axis_index("s") * info.num_cores + lax.axis_index("c")
        base = wid * per_w
        sems = (sem0, sem1)

        def fire(b, par):
            r0 = base + b * _GB
            pltpu.sync_copy(idx_hbm.at[pl.ds(r0, _GB)], idx_v.at[par])
            pltpu.sync_copy(w_hbm.at[pl.ds(r0, _GB)], w_v.at[par])
            for g in range(_GB):
                pltpu.async_copy(
                    table_hbm.at[idx_v.at[par, g]], rows_v.at[par, g], sems[par])

        def drain(par):
            for g in range(_GB):
                pltpu.make_async_copy(
                    table_hbm.at[pl.ds(0, _KC)], rows_v.at[par, g],
                    sems[par]).wait()

        def compute_store(b, par):
            r0 = base + b * _GB
            zero = jnp.zeros((16,), jnp.float32)
            for g in range(_GB):
                rows = rows_v.at[par, g]

                def cstep(c, acc):
                    wvec = w_v[par, g, pl.ds(c * 16, 16)]
                    acc = list(acc)
                    for j in range(16):
                        s = wvec[j]
                        k = c * 16 + j
                        t = j % 4
                        acc[2 * t] = acc[2 * t] + rows[k, pl.ds(0, 16)] * s
                        acc[2 * t + 1] = acc[2 * t + 1] + rows[k, pl.ds(16, 16)] * s
                    return tuple(acc)

                acc = lax.fori_loop(0, _KC // 16, cstep, (zero,) * 8)
                out_v[par, g, pl.ds(0, 16)] = (acc[0] + acc[2]) + (acc[4] + acc[6])
                out_v[par, g, pl.ds(16, 16)] = (acc[1] + acc[3]) + (acc[5] + acc[7])
            pltpu.sync_copy(out_v.at[par], out_hbm.at[pl.ds(r0, _GB)])

        fire(0, 0)
        nouter = nb // 2

        def outer(i, _):
            b0 = 2 * i
            fire(b0 + 1, 1)
            drain(0)
            compute_store(b0, 0)

            @pl.when(i + 1 < nouter)
            def _():
                fire(b0 + 2, 0)

            drain(1)
            compute_store(b0 + 1, 1)
            return 0

        lax.fori_loop(0, nouter, outer, 0)

    return run(idx, w, table)


def kernel(query, value, spatial_shapes, reference_points, query_pos,
           level_start_index, Wv, bv, Wo, bo, Wa, ba):
    del spatial_shapes, level_start_index
    bs, nq, _ = query.shape
    nv = value.shape[1]
    q2 = query.reshape(nq, _EMBED)
    qp2 = query_pos.reshape(nq, _EMBED)

    table = pl.pallas_call(
        _vproj_body,
        grid=(pl.cdiv(nv, _VBLK),),
        in_specs=[
            pl.BlockSpec((_VBLK, _EMBED), lambda i: (i, 0)),
            pl.BlockSpec((_EMBED, _EMBED), lambda i: (0, 0)),
            pl.BlockSpec((1, _EMBED), lambda i: (0, 0)),
        ],
        out_specs=pl.BlockSpec((_VBLK, _EMBED), lambda i: (i, 0)),
        out_shape=jax.ShapeDtypeStruct((nv, _EMBED), jnp.float32),
    )(value.reshape(nv, _EMBED), Wv, bv.reshape(1, _EMBED))
    table = table.reshape(nv * _HEADS, _HD)

    idx, w = _sampling_params(q2, qp2, reference_points, Wo, bo, Wa, ba, nq)
    npair = nq * _HEADS
    idx = idx.reshape(npair, _KC)
    w = w.reshape(npair, _KC)

    out = _gather_reduce(idx, w, table, npair)
    return out.reshape(bs, nq, _EMBED)


# speedup vs baseline: 3086.5447x; 1.0898x over previous
import functools

import jax
import jax.numpy as jnp
import numpy as np
from jax import lax
from jax.experimental import pallas as pl
from jax.experimental.pallas import tpu as pltpu
from jax.experimental.pallas import tpu_sc as plsc

_EMBED = 256
_HEADS = 8
_LEVELS = 4
_POINTS = 8
_ZA = 4
_HD = _EMBED // _HEADS
_SHAPES = ((116, 200), (58, 100), (29, 50), (15, 25))
_NV = sum(h * w for h, w in _SHAPES)
_K = _LEVELS * _POINTS
_KC = 4 * _K

_VBLK = 1024
_QBLK = 1000


def _col_consts():
    k = np.arange(_HEADS * _K)
    h = k // _K
    l = (k % _K) // _POINTS
    p = k % _POINTS
    z = p % _ZA
    wf = np.array([s[1] for s in _SHAPES], np.float32)[l]
    hf = np.array([s[0] for s in _SHAPES], np.float32)[l]
    sizes = [s[0] * s[1] for s in _SHAPES]
    offs = np.concatenate([[0], np.cumsum(sizes)[:-1]]).astype(np.int32)
    ls = offs[l]
    sx = np.zeros((2 * _ZA, _HEADS * _K), np.float32)
    sy = np.zeros((2 * _ZA, _HEADS * _K), np.float32)
    sx[2 * z, k] = 1.0
    sy[2 * z + 1, k] = 1.0
    ones = (k[:, None] // _K == k[None, :] // _K).astype(np.float32)
    return (wf.astype(np.float32), hf.astype(np.float32), ls, h.astype(np.int32),
            wf.astype(np.int32), hf.astype(np.int32), sx, sy, ones)


def _vproj_body(val_ref, wv_ref, bv_ref, out_ref):
    out_ref[...] = (
        jnp.dot(val_ref[...], wv_ref[...], preferred_element_type=jnp.float32, precision=jax.lax.Precision.HIGHEST)
        + bv_ref[...]
    )


def _sample_body(q_ref, qp_ref, rp_ref, wox_ref, box_ref, woy_ref, boy_ref,
                 wa_ref, ba_ref, ones_ref, sx_ref, sy_ref, wcol_ref, hcol_ref,
                 ls_ref, hid_ref, wi_ref, hi_ref, idx_ref, w_ref):
    q = q_ref[...] + qp_ref[...]
    sox = jnp.dot(q, wox_ref[...], preferred_element_type=jnp.float32, precision=jax.lax.Precision.HIGHEST) + box_ref[...]
    soy = jnp.dot(q, woy_ref[...], preferred_element_type=jnp.float32, precision=jax.lax.Precision.HIGHEST) + boy_ref[...]
    al = jnp.dot(q, wa_ref[...], preferred_element_type=jnp.float32, precision=jax.lax.Precision.HIGHEST) + ba_ref[...]
    m = jnp.max(al, axis=-1, keepdims=True)
    e = jnp.exp(al - m)
    denom = jnp.dot(e, ones_ref[...], preferred_element_type=jnp.float32, precision=jax.lax.Precision.HIGHEST)
    aw = e / denom
    rpx = jnp.dot(rp_ref[...], sx_ref[...], preferred_element_type=jnp.float32, precision=jax.lax.Precision.HIGHEST)
    rpy = jnp.dot(rp_ref[...], sy_ref[...], preferred_element_type=jnp.float32, precision=jax.lax.Precision.HIGHEST)
    px = rpx * wcol_ref[...] + sox - 0.5
    py = rpy * hcol_ref[...] + soy - 0.5
    x0 = jnp.floor(px)
    y0 = jnp.floor(py)
    fx = px - x0
    fy = py - y0
    x0i = x0.astype(jnp.int32)
    y0i = y0.astype(jnp.int32)
    wi = wi_ref[...]
    hi = hi_ref[...]
    ls = ls_ref[...]
    hid = hid_ref[...]
    corners = (
        (0, 0, (1.0 - fx) * (1.0 - fy)),
        (1, 0, fx * (1.0 - fy)),
        (0, 1, (1.0 - fx) * fy),
        (1, 1, fx * fy),
    )
    for c, (dx, dy, wbil) in enumerate(corners):
        xi = x0i + dx
        yi = y0i + dy
        valid = (xi >= 0) & (xi < wi) & (yi >= 0) & (yi < hi)
        xc = jnp.clip(xi, 0, wi - 1)
        yc = jnp.clip(yi, 0, hi - 1)
        gidx = (ls + yc * wi + xc) * _HEADS + hid
        wgt = aw * wbil * valid.astype(jnp.float32)
        idx_ref[c] = gidx
        w_ref[c] = wgt


def _sampling_params(query, query_pos, reference_points, Wo, bo, Wa, ba, nq):
    wf, hf, ls, hid, wi, hi, sx, sy, ones = _col_consts()
    wox = Wo[:, 0::2]
    woy = Wo[:, 1::2]
    box = bo[0::2].reshape(1, -1)
    boy = bo[1::2].reshape(1, -1)
    rp_flat = reference_points.reshape(nq, 2 * _ZA)
    n256 = _HEADS * _K
    row = lambda a: jnp.asarray(a).reshape(1, n256)
    grid = pl.cdiv(nq, _QBLK)
    bcast = lambda shape: pl.BlockSpec(shape, lambda i: (0,) * len(shape))
    return pl.pallas_call(
        _sample_body,
        grid=(grid,),
        in_specs=[
            pl.BlockSpec((_QBLK, _EMBED), lambda i: (i, 0)),
            pl.BlockSpec((_QBLK, _EMBED), lambda i: (i, 0)),
            pl.BlockSpec((_QBLK, 2 * _ZA), lambda i: (i, 0)),
            bcast((_EMBED, n256)), bcast((1, n256)),
            bcast((_EMBED, n256)), bcast((1, n256)),
            bcast((_EMBED, n256)), bcast((1, n256)),
            bcast((n256, n256)),
            bcast((2 * _ZA, n256)), bcast((2 * _ZA, n256)),
            bcast((1, n256)), bcast((1, n256)),
            bcast((1, n256)), bcast((1, n256)),
            bcast((1, n256)), bcast((1, n256)),
        ],
        out_specs=[
            pl.BlockSpec((4, _QBLK, _HEADS * _K), lambda i: (0, i, 0)),
            pl.BlockSpec((4, _QBLK, _HEADS * _K), lambda i: (0, i, 0)),
        ],
        out_shape=[
            jax.ShapeDtypeStruct((4, nq, _HEADS * _K), jnp.int32),
            jax.ShapeDtypeStruct((4, nq, _HEADS * _K), jnp.float32),
        ],
    )(query, query_pos, rp_flat, wox, box, woy, boy,
      jnp.asarray(Wa), jnp.asarray(ba).reshape(1, n256), jnp.asarray(ones),
      jnp.asarray(sx), jnp.asarray(sy), row(wf), row(hf),
      row(ls), row(hid), row(wi), row(hi))


_GB = 10


def _gather_reduce(idx, w, table, nq):
    info = plsc.get_sparse_core_info()
    nw = info.num_cores * info.num_subcores
    nchunk = nw // _HEADS
    per_w = nq // nchunk
    nb = per_w // _GB
    mesh = plsc.VectorSubcoreMesh(
        core_axis_name="c", subcore_axis_name="s",
        num_cores=info.num_cores, num_subcores=info.num_subcores)

    @functools.partial(
        pl.kernel,
        out_type=jax.ShapeDtypeStruct((nq, _EMBED), jnp.float32),
        mesh=mesh,
        scratch_types=[
            pltpu.VMEM((2, _GB, _KC), jnp.int32),
            pltpu.VMEM((2, _GB, _KC), jnp.float32),
            pltpu.VMEM((2, _GB, _KC, _HD), jnp.float32),
            pltpu.VMEM((2, _GB, _HD), jnp.float32),
            pltpu.SemaphoreType.DMA,
            pltpu.SemaphoreType.DMA,
        ],
        compiler_params=pltpu.CompilerParams(use_tc_tiling_on_sc=False),
    )
    def run(idx_hbm, w_hbm, table_hbm, out_hbm, idx_v, w_v, rows_v, out_v,
            sem0, sem1):
        wid = lax.axis_index("s") * info.num_cores + lax.axis_index("c")
        head = wid % _HEADS
        qbase = (wid // _HEADS) * per_w
        h32 = head * _HD
        sems = (sem0, sem1)

        def fire(b, par):
            q0 = qbase + b * _GB
            for c in range(4):
                pltpu.sync_copy(
                    idx_hbm.at[c, pl.ds(q0, _GB), pl.ds(h32, _K)],
                    idx_v.at[par, :, pl.ds(c * _K, _K)])
                pltpu.sync_copy(
                    w_hbm.at[c, pl.ds(q0, _GB), pl.ds(h32, _K)],
                    w_v.at[par, :, pl.ds(c * _K, _K)])
            for g in range(_GB):
                pltpu.async_copy(
                    table_hbm.at[idx_v.at[par, g]], rows_v.at[par, g], sems[par])

        def drain(par):
            for g in range(_GB):
                pltpu.make_async_copy(
                    table_hbm.at[pl.ds(0, _KC)], rows_v.at[par, g],
                    sems[par]).wait()

        def compute_store(b, par):
            q0 = qbase + b * _GB
            zero = jnp.zeros((16,), jnp.float32)
            for g in range(_GB):
                rows = rows_v.at[par, g]

                def cstep(c, acc):
                    wvec = w_v[par, g, pl.ds(c * 16, 16)]
                    acc = list(acc)
                    for j in range(16):
                        s = wvec[j]
                        k = c * 16 + j
                        t = j % 4
                        acc[2 * t] = acc[2 * t] + rows[k, pl.ds(0, 16)] * s
                        acc[2 * t + 1] = acc[2 * t + 1] + rows[k, pl.ds(16, 16)] * s
                    return tuple(acc)

                acc = lax.fori_loop(0, _KC // 16, cstep, (zero,) * 8)
                out_v[par, g, pl.ds(0, 16)] = (acc[0] + acc[2]) + (acc[4] + acc[6])
                out_v[par, g, pl.ds(16, 16)] = (acc[1] + acc[3]) + (acc[5] + acc[7])
            pltpu.sync_copy(
                out_v.at[par], out_hbm.at[pl.ds(q0, _GB), pl.ds(h32, _HD)])

        fire(0, 0)
        nouter = nb // 2

        def outer(i, _):
            b0 = 2 * i
            fire(b0 + 1, 1)
            drain(0)
            compute_store(b0, 0)

            @pl.when(i + 1 < nouter)
            def _():
                fire(b0 + 2, 0)

            drain(1)
            compute_store(b0 + 1, 1)
            return 0

        lax.fori_loop(0, nouter, outer, 0)

    return run(idx, w, table)


def kernel(query, value, spatial_shapes, reference_points, query_pos,
           level_start_index, Wv, bv, Wo, bo, Wa, ba):
    del spatial_shapes, level_start_index
    bs, nq, _ = query.shape
    nv = value.shape[1]
    q2 = query.reshape(nq, _EMBED)
    qp2 = query_pos.reshape(nq, _EMBED)

    table = pl.pallas_call(
        _vproj_body,
        grid=(pl.cdiv(nv, _VBLK),),
        in_specs=[
            pl.BlockSpec((_VBLK, _EMBED), lambda i: (i, 0)),
            pl.BlockSpec((_EMBED, _EMBED), lambda i: (0, 0)),
            pl.BlockSpec((1, _EMBED), lambda i: (0, 0)),
        ],
        out_specs=pl.BlockSpec((_VBLK, _EMBED), lambda i: (i, 0)),
        out_shape=jax.ShapeDtypeStruct((nv, _EMBED), jnp.float32),
    )(value.reshape(nv, _EMBED), Wv, bv.reshape(1, _EMBED))
    table = table.reshape(nv * _HEADS, _HD)

    idx, w = _sampling_params(q2, qp2, reference_points, Wo, bo, Wa, ba, nq)
    out = _gather_reduce(idx, w, table, nq)
    return out.reshape(bs, nq, _EMBED)


# speedup vs baseline: 3379.5076x; 1.0949x over previous
import functools

import jax
import jax.numpy as jnp
import numpy as np
from jax import lax
from jax.experimental import pallas as pl
from jax.experimental.pallas import tpu as pltpu
from jax.experimental.pallas import tpu_sc as plsc

_EMBED = 256
_HEADS = 8
_LEVELS = 4
_POINTS = 8
_ZA = 4
_HD = _EMBED // _HEADS
_SHAPES = ((116, 200), (58, 100), (29, 50), (15, 25))
_NV = sum(h * w for h, w in _SHAPES)
_K = _LEVELS * _POINTS
_KC = 4 * _K

_VBLK = 1024
_QBLK = 1000


def _col_consts():
    k = np.arange(_HEADS * _K)
    h = k // _K
    l = (k % _K) // _POINTS
    p = k % _POINTS
    z = p % _ZA
    wf = np.array([s[1] for s in _SHAPES], np.float32)[l]
    hf = np.array([s[0] for s in _SHAPES], np.float32)[l]
    sizes = [s[0] * s[1] for s in _SHAPES]
    offs = np.concatenate([[0], np.cumsum(sizes)[:-1]]).astype(np.int32)
    ls = offs[l]
    sx = np.zeros((2 * _ZA, _HEADS * _K), np.float32)
    sy = np.zeros((2 * _ZA, _HEADS * _K), np.float32)
    sx[2 * z, k] = 1.0
    sy[2 * z + 1, k] = 1.0
    ones = (k[:, None] // _K == k[None, :] // _K).astype(np.float32)
    return (wf.astype(np.float32), hf.astype(np.float32), ls, h.astype(np.int32),
            wf.astype(np.int32), hf.astype(np.int32), sx, sy, ones)


def _vproj_body(val_ref, wv_ref, bv_ref, out_ref):
    out_ref[...] = (
        jnp.dot(val_ref[...], wv_ref[...], preferred_element_type=jnp.float32, precision=jax.lax.Precision.HIGHEST)
        + bv_ref[...]
    )


def _sample_body(q_ref, qp_ref, rp_ref, wox_ref, box_ref, woy_ref, boy_ref,
                 wa_ref, ba_ref, ones_ref, sx_ref, sy_ref, wcol_ref, hcol_ref,
                 ls_ref, hid_ref, wi_ref, hi_ref, idx_ref, w_ref):
    q = q_ref[...] + qp_ref[...]
    sox = jnp.dot(q, wox_ref[...], preferred_element_type=jnp.float32, precision=jax.lax.Precision.HIGHEST) + box_ref[...]
    soy = jnp.dot(q, woy_ref[...], preferred_element_type=jnp.float32, precision=jax.lax.Precision.HIGHEST) + boy_ref[...]
    al = jnp.dot(q, wa_ref[...], preferred_element_type=jnp.float32, precision=jax.lax.Precision.HIGHEST) + ba_ref[...]
    m = jnp.max(al, axis=-1, keepdims=True)
    e = jnp.exp(al - m)
    denom = jnp.dot(e, ones_ref[...], preferred_element_type=jnp.float32, precision=jax.lax.Precision.HIGHEST)
    aw = e / denom
    rpx = jnp.dot(rp_ref[...], sx_ref[...], preferred_element_type=jnp.float32, precision=jax.lax.Precision.HIGHEST)
    rpy = jnp.dot(rp_ref[...], sy_ref[...], preferred_element_type=jnp.float32, precision=jax.lax.Precision.HIGHEST)
    px = rpx * wcol_ref[...] + sox - 0.5
    py = rpy * hcol_ref[...] + soy - 0.5
    x0 = jnp.floor(px)
    y0 = jnp.floor(py)
    fx = px - x0
    fy = py - y0
    x0i = x0.astype(jnp.int32)
    y0i = y0.astype(jnp.int32)
    wi = wi_ref[...]
    hi = hi_ref[...]
    ls = ls_ref[...]
    hid = hid_ref[...]
    corners = (
        (0, 0, (1.0 - fx) * (1.0 - fy)),
        (1, 0, fx * (1.0 - fy)),
        (0, 1, (1.0 - fx) * fy),
        (1, 1, fx * fy),
    )
    for c, (dx, dy, wbil) in enumerate(corners):
        xi = x0i + dx
        yi = y0i + dy
        valid = (xi >= 0) & (xi < wi) & (yi >= 0) & (yi < hi)
        xc = jnp.clip(xi, 0, wi - 1)
        yc = jnp.clip(yi, 0, hi - 1)
        gidx = (ls + yc * wi + xc) * _HEADS + hid
        wgt = aw * wbil * valid.astype(jnp.float32)
        idx_ref[c] = gidx
        w_ref[c] = wgt


def _sampling_params(query, query_pos, reference_points, Wo, bo, Wa, ba, nq):
    wf, hf, ls, hid, wi, hi, sx, sy, ones = _col_consts()
    wox = Wo[:, 0::2]
    woy = Wo[:, 1::2]
    box = bo[0::2].reshape(1, -1)
    boy = bo[1::2].reshape(1, -1)
    rp_flat = reference_points.reshape(nq, 2 * _ZA)
    n256 = _HEADS * _K
    row = lambda a: jnp.asarray(a).reshape(1, n256)
    grid = pl.cdiv(nq, _QBLK)
    bcast = lambda shape: pl.BlockSpec(shape, lambda i: (0,) * len(shape))
    return pl.pallas_call(
        _sample_body,
        grid=(grid,),
        in_specs=[
            pl.BlockSpec((_QBLK, _EMBED), lambda i: (i, 0)),
            pl.BlockSpec((_QBLK, _EMBED), lambda i: (i, 0)),
            pl.BlockSpec((_QBLK, 2 * _ZA), lambda i: (i, 0)),
            bcast((_EMBED, n256)), bcast((1, n256)),
            bcast((_EMBED, n256)), bcast((1, n256)),
            bcast((_EMBED, n256)), bcast((1, n256)),
            bcast((n256, n256)),
            bcast((2 * _ZA, n256)), bcast((2 * _ZA, n256)),
            bcast((1, n256)), bcast((1, n256)),
            bcast((1, n256)), bcast((1, n256)),
            bcast((1, n256)), bcast((1, n256)),
        ],
        out_specs=[
            pl.BlockSpec((4, _QBLK, _HEADS * _K), lambda i: (0, i, 0)),
            pl.BlockSpec((4, _QBLK, _HEADS * _K), lambda i: (0, i, 0)),
        ],
        out_shape=[
            jax.ShapeDtypeStruct((4, nq, _HEADS * _K), jnp.int32),
            jax.ShapeDtypeStruct((4, nq, _HEADS * _K), jnp.float32),
        ],
    )(query, query_pos, rp_flat, wox, box, woy, boy,
      jnp.asarray(Wa), jnp.asarray(ba).reshape(1, n256), jnp.asarray(ones),
      jnp.asarray(sx), jnp.asarray(sy), row(wf), row(hf),
      row(ls), row(hid), row(wi), row(hi))


_GB = 10


def _gather_reduce(idx, w, table, npair):
    info = plsc.get_sparse_core_info()
    nw = info.num_cores * info.num_subcores
    per_w = npair // nw
    nb = per_w // _GB
    mesh = plsc.VectorSubcoreMesh(
        core_axis_name="c", subcore_axis_name="s",
        num_cores=info.num_cores, num_subcores=info.num_subcores)

    @functools.partial(
        pl.kernel,
        out_type=jax.ShapeDtypeStruct((npair, _HD), jnp.float32),
        mesh=mesh,
        scratch_types=[
            pltpu.VMEM((2, _GB, _KC), jnp.int32),
            pltpu.VMEM((2, _GB, _KC), jnp.float32),
            pltpu.VMEM((2, _GB, _KC, _HD), jnp.float32),
            pltpu.VMEM((2, _GB, _HD), jnp.float32),
            pltpu.SemaphoreType.DMA,
            pltpu.SemaphoreType.DMA,
        ],
        compiler_params=pltpu.CompilerParams(use_tc_tiling_on_sc=False),
    )
    def run(idx_hbm, w_hbm, table_hbm, out_hbm, idx_v, w_v, rows_v, out_v,
            sem0, sem1):
        wid = lax.axis_index("s") * info.num_cores + lax.axis_index("c")
        base = wid * per_w
        sems = (sem0, sem1)

        def fire(b, par):
            r0 = base + b * _GB
            pltpu.sync_copy(idx_hbm.at[pl.ds(r0, _GB)], idx_v.at[par])
            pltpu.sync_copy(w_hbm.at[pl.ds(r0, _GB)], w_v.at[par])
            for g in range(_GB):
                pltpu.async_copy(
                    table_hbm.at[idx_v.at[par, g]], rows_v.at[par, g], sems[par])

        def drain(par):
            for g in range(_GB):
                pltpu.make_async_copy(
                    table_hbm.at[pl.ds(0, _KC)], rows_v.at[par, g],
                    sems[par]).wait()

        def compute_store(b, par):
            r0 = base + b * _GB
            zero = jnp.zeros((16,), jnp.float32)
            for g in range(_GB):
                rows = rows_v.at[par, g]

                def cstep(c, acc):
                    wvec = w_v[par, g, pl.ds(c * 16, 16)]
                    acc = list(acc)
                    for j in range(16):
                        s = wvec[j]
                        k = c * 16 + j
                        t = j % 4
                        acc[2 * t] = acc[2 * t] + rows[k, pl.ds(0, 16)] * s
                        acc[2 * t + 1] = acc[2 * t + 1] + rows[k, pl.ds(16, 16)] * s
                    return tuple(acc)

                acc = lax.fori_loop(0, _KC // 16, cstep, (zero,) * 8)
                out_v[par, g, pl.ds(0, 16)] = (acc[0] + acc[2]) + (acc[4] + acc[6])
                out_v[par, g, pl.ds(16, 16)] = (acc[1] + acc[3]) + (acc[5] + acc[7])
            pltpu.sync_copy(out_v.at[par], out_hbm.at[pl.ds(r0, _GB)])

        fire(0, 0)
        nouter = nb // 2

        def outer(i, _):
            b0 = 2 * i
            fire(b0 + 1, 1)
            drain(0)
            compute_store(b0, 0)

            @pl.when(i + 1 < nouter)
            def _():
                fire(b0 + 2, 0)

            drain(1)
            compute_store(b0 + 1, 1)
            return 0

        lax.fori_loop(0, nouter, outer, 0)

    return run(idx, w, table)


def kernel(query, value, spatial_shapes, reference_points, query_pos,
           level_start_index, Wv, bv, Wo, bo, Wa, ba):
    del spatial_shapes, level_start_index
    bs, nq, _ = query.shape
    nv = value.shape[1]
    q2 = query.reshape(nq, _EMBED)
    qp2 = query_pos.reshape(nq, _EMBED)

    table = pl.pallas_call(
        _vproj_body,
        grid=(pl.cdiv(nv, _VBLK),),
        in_specs=[
            pl.BlockSpec((_VBLK, _EMBED), lambda i: (i, 0)),
            pl.BlockSpec((_EMBED, _EMBED), lambda i: (0, 0)),
            pl.BlockSpec((1, _EMBED), lambda i: (0, 0)),
        ],
        out_specs=pl.BlockSpec((_VBLK, _EMBED), lambda i: (i, 0)),
        out_shape=jax.ShapeDtypeStruct((nv, _EMBED), jnp.float32),
    )(value.reshape(nv, _EMBED), Wv, bv.reshape(1, _EMBED))
    table = table.reshape(nv * _HEADS, _HD)

    idx, w = _sampling_params(q2, qp2, reference_points, Wo, bo, Wa, ba, nq)
    npair = nq * _HEADS
    idx = jnp.transpose(idx.reshape(4, nq, _HEADS, _K), (1, 2, 0, 3)).reshape(npair, _KC)
    w = jnp.transpose(w.reshape(4, nq, _HEADS, _K), (1, 2, 0, 3)).reshape(npair, _KC)
    out = _gather_reduce(idx, w, table, npair)
    return out.reshape(bs, nq, _EMBED)
